# baseline (speedup 1.0000x reference)
import functools

import jax
import jax.numpy as jnp
from jax import lax
from jax.experimental import pallas as pl
from jax.experimental.pallas import tpu as pltpu
from jax.experimental.pallas import tpu_sc as plsc

F = 26
B = 16384
D = 32
V = 2600000
TOTAL = B * F
NC = 2
NS = 16
NW = NC * NS

PER_W = TOTAL // NW
CHUNK = 3328
NCH = PER_W // CHUNK
NV = CHUNK // 16
NG = NW * NCH

VR = V // 4
NT_FULL = V // 128
T_LAST = NT_FULL - 1
NI = 636
TAILR = V - NT_FULL * 128


def _relayout_tile(tsrc, tstage, iota16):
    for R in range(32):
        for L0 in range(0, 128, 16):
            d_vec = iota16 + (L0 % 32)
            rl = 4 * R + L0 // 32
            rl_vec = jnp.full((16,), rl, jnp.int32)
            tstage[R, pl.ds(L0, 16)] = plsc.load_gather(tsrc, [d_vec, rl_vec])


def _k1_body(tT_hbm, tail_hbm, out_hbm, tsrc0, tsrc1, tst0, tst1, si0, si1, so0, so1):
    wid = lax.axis_index("s") * NC + lax.axis_index("c")
    iota16 = lax.iota(jnp.int32, 16)
    tsrc = (tsrc0, tsrc1)
    tst = (tst0, tst1)
    sin = (si0, si1)
    sout = (so0, so1)

    def tile_of(i):
        return jnp.minimum(wid + 32 * i, T_LAST)

    def issue_in(i, b):
        t = tile_of(i)
        pltpu.async_copy(
            tT_hbm.at[:, pl.ds(128 * t, 128)], tsrc[b], sin[b]
        )

    def wait_in(b):
        pltpu.make_async_copy(
            tT_hbm.at[:, pl.ds(0, 128)], tsrc[b], sin[b]
        ).wait()

    def issue_out(i, b):
        t = tile_of(i)
        pltpu.async_copy(tst[b], out_hbm.at[pl.ds(32 * t, 32)], sout[b])

    def wait_out(b):
        pltpu.make_async_copy(
            tst[b], out_hbm.at[pl.ds(0, 32)], sout[b]
        ).wait()

    issue_in(0, 0)
    issue_in(1, 1)
    for b in (0, 1):
        wait_in(b)
        _relayout_tile(tsrc[b], tst[b], iota16)
        issue_out(b, b)
        issue_in(b + 2, b)

    def pair_body(j):
        for b in (0, 1):
            i = 2 * j + b
            wait_in(b)
            wait_out(b)
            _relayout_tile(tsrc[b], tst[b], iota16)
            issue_out(i, b)
            issue_in(i + 2, b)

    pl.loop(1, NI // 2)(pair_body)

    for b in (0, 1):
        wait_in(b)
        wait_out(b)

    @pl.when(wid == 0)
    def _tail():
        pltpu.sync_copy(tail_hbm, out_hbm.at[pl.ds(32 * NT_FULL, TAILR // 4)])


@jax.jit
def _relayout_call(tableT, tail16):
    mesh = plsc.VectorSubcoreMesh(
        core_axis_name="c", subcore_axis_name="s", num_cores=NC, num_subcores=NS
    )
    return pl.kernel(
        _k1_body,
        out_type=jax.ShapeDtypeStruct((VR, 128), jnp.float32),
        mesh=mesh,
        scratch_types=[
            pltpu.VMEM((32, 128), jnp.float32),
            pltpu.VMEM((32, 128), jnp.float32),
            pltpu.VMEM((32, 128), jnp.float32),
            pltpu.VMEM((32, 128), jnp.float32),
            pltpu.SemaphoreType.DMA,
            pltpu.SemaphoreType.DMA,
            pltpu.SemaphoreType.DMA,
            pltpu.SemaphoreType.DMA,
        ],
        compiler_params=pltpu.CompilerParams(
            use_tc_tiling_on_sc=True, needs_layout_passes=False
        ),
    )(tableT, tail16)


def _emb_body(x_hbm, off_hbm, table_hbm, out_hbm, obuf, xbuf, rows, sem):
    wid = lax.axis_index("s") * NC + lax.axis_index("c")
    pltpu.sync_copy(off_hbm, obuf)

    def chunk_body(c):
        g = wid * NCH + c
        pltpu.sync_copy(x_hbm.at[g], xbuf)
        for v in range(NV):
            sl = pl.ds(v * 16, 16)
            xbuf[sl] = xbuf[sl] + obuf[sl]
        pltpu.async_copy(table_hbm.at[xbuf], rows, sem).wait()
        pltpu.sync_copy(rows, out_hbm.at[g])

    pl.loop(0, NCH)(chunk_body)


@jax.jit
def _emb_call(x2, off1, table):
    mesh = plsc.VectorSubcoreMesh(
        core_axis_name="c", subcore_axis_name="s", num_cores=NC, num_subcores=NS
    )
    return pl.kernel(
        _emb_body,
        out_type=jax.ShapeDtypeStruct((NG, CHUNK, D), jnp.float32),
        mesh=mesh,
        scratch_types=[
            pltpu.VMEM((CHUNK,), jnp.int32),
            pltpu.VMEM((CHUNK,), jnp.int32),
            pltpu.VMEM((CHUNK, D), jnp.float32),
            pltpu.SemaphoreType.DMA,
        ],
        compiler_params=pltpu.CompilerParams(use_tc_tiling_on_sc=False),
    )(x2, off1, table)


def kernel(x, table, offsets):
    tail16 = table[V - TAILR:, :].reshape(TAILR // 4, 128)
    t128 = _relayout_call(table.T, tail16)
    tableR = t128.reshape(V, D)
    x2 = x.reshape(NG, CHUNK)
    off1 = jnp.tile(offsets, CHUNK // F)
    out = _emb_call(x2, off1, tableR)
    return out.reshape(B, F * D)


# speedup vs baseline: 1.8534x; 1.8534x over previous
import functools

import jax
import jax.numpy as jnp
from jax import lax
from jax.experimental import pallas as pl
from jax.experimental.pallas import tpu as pltpu
from jax.experimental.pallas import tpu_sc as plsc

F = 26
B = 16384
D = 32
V = 2600000
TOTAL = B * F
NC = 2
NS = 16
NW = NC * NS

PER_W = TOTAL // NW
CHUNK = 3328
NCH = PER_W // CHUNK
NV = CHUNK // 16
NG = NW * NCH

VR = V // 4
NT_FULL = V // 128
T_LAST = NT_FULL - 1
NI = 636
TAILR = V - NT_FULL * 128


def _relayout_tile(tsrc, tstage, iota16):
    @plsc.parallel_loop(0, 32, unroll=4)
    def _row(R):
        for L0 in range(0, 128, 16):
            d_vec = iota16 + (L0 % 32)
            rl_vec = jnp.full((16,), L0 // 32, jnp.int32) + 4 * R
            tstage[R, pl.ds(L0, 16)] = plsc.load_gather(tsrc, [d_vec, rl_vec])


def _k1_body(tT_hbm, tail_hbm, out_hbm, tsrc0, tsrc1, tst0, tst1, si0, si1, so0, so1):
    wid = lax.axis_index("s") * NC + lax.axis_index("c")
    iota16 = lax.iota(jnp.int32, 16)
    tsrc = (tsrc0, tsrc1)
    tst = (tst0, tst1)
    sin = (si0, si1)
    sout = (so0, so1)

    def tile_of(i):
        return jnp.minimum(wid + 32 * i, T_LAST)

    def issue_in(i, b):
        t = tile_of(i)
        pltpu.async_copy(
            tT_hbm.at[:, pl.ds(128 * t, 128)], tsrc[b], sin[b]
        )

    def wait_in(b):
        pltpu.make_async_copy(
            tT_hbm.at[:, pl.ds(0, 128)], tsrc[b], sin[b]
        ).wait()

    def issue_out(i, b):
        t = tile_of(i)
        pltpu.async_copy(tst[b], out_hbm.at[pl.ds(32 * t, 32)], sout[b])

    def wait_out(b):
        pltpu.make_async_copy(
            tst[b], out_hbm.at[pl.ds(0, 32)], sout[b]
        ).wait()

    issue_in(0, 0)
    issue_in(1, 1)
    for b in (0, 1):
        wait_in(b)
        _relayout_tile(tsrc[b], tst[b], iota16)
        issue_out(b, b)
        issue_in(b + 2, b)

    def pair_body(j):
        for b in (0, 1):
            i = 2 * j + b
            wait_in(b)
            wait_out(b)
            _relayout_tile(tsrc[b], tst[b], iota16)
            issue_out(i, b)
            issue_in(i + 2, b)

    pl.loop(1, NI // 2)(pair_body)

    for b in (0, 1):
        wait_in(b)
        wait_out(b)

    @pl.when(wid == 0)
    def _tail():
        pltpu.sync_copy(tail_hbm, out_hbm.at[pl.ds(32 * NT_FULL, TAILR // 4)])


@jax.jit
def _relayout_call(tableT, tail16):
    mesh = plsc.VectorSubcoreMesh(
        core_axis_name="c", subcore_axis_name="s", num_cores=NC, num_subcores=NS
    )
    return pl.kernel(
        _k1_body,
        out_type=jax.ShapeDtypeStruct((VR, 128), jnp.float32),
        mesh=mesh,
        scratch_types=[
            pltpu.VMEM((32, 128), jnp.float32),
            pltpu.VMEM((32, 128), jnp.float32),
            pltpu.VMEM((32, 128), jnp.float32),
            pltpu.VMEM((32, 128), jnp.float32),
            pltpu.SemaphoreType.DMA,
            pltpu.SemaphoreType.DMA,
            pltpu.SemaphoreType.DMA,
            pltpu.SemaphoreType.DMA,
        ],
        compiler_params=pltpu.CompilerParams(
            use_tc_tiling_on_sc=True, needs_layout_passes=False
        ),
    )(tableT, tail16)


def _emb_body(x_hbm, off_hbm, table_hbm, out_hbm, obuf, xbuf, rows, sem):
    wid = lax.axis_index("s") * NC + lax.axis_index("c")
    pltpu.sync_copy(off_hbm, obuf)

    def chunk_body(c):
        g = wid * NCH + c
        pltpu.sync_copy(x_hbm.at[g], xbuf)
        for v in range(NV):
            sl = pl.ds(v * 16, 16)
            xbuf[sl] = xbuf[sl] + obuf[sl]
        pltpu.async_copy(table_hbm.at[xbuf], rows, sem).wait()
        pltpu.sync_copy(rows, out_hbm.at[g])

    pl.loop(0, NCH)(chunk_body)


@jax.jit
def _emb_call(x2, off1, table):
    mesh = plsc.VectorSubcoreMesh(
        core_axis_name="c", subcore_axis_name="s", num_cores=NC, num_subcores=NS
    )
    return pl.kernel(
        _emb_body,
        out_type=jax.ShapeDtypeStruct((NG, CHUNK, D), jnp.float32),
        mesh=mesh,
        scratch_types=[
            pltpu.VMEM((CHUNK,), jnp.int32),
            pltpu.VMEM((CHUNK,), jnp.int32),
            pltpu.VMEM((CHUNK, D), jnp.float32),
            pltpu.SemaphoreType.DMA,
        ],
        compiler_params=pltpu.CompilerParams(use_tc_tiling_on_sc=False),
    )(x2, off1, table)


def kernel(x, table, offsets):
    tail16 = table[V - TAILR:, :].reshape(TAILR // 4, 128)
    t128 = _relayout_call(table.T, tail16)
    tableR = t128.reshape(V, D)
    x2 = x.reshape(NG, CHUNK)
    off1 = jnp.tile(offsets, CHUNK // F)
    out = _emb_call(x2, off1, tableR)
    return out.reshape(B, F * D)


# speedup vs baseline: 1.8589x; 1.0030x over previous
import functools

import jax
import jax.numpy as jnp
from jax import lax
from jax.experimental import pallas as pl
from jax.experimental.pallas import tpu as pltpu
from jax.experimental.pallas import tpu_sc as plsc

F = 26
B = 16384
D = 32
V = 2600000
TOTAL = B * F
NC = 2
NS = 16
NW = NC * NS

PER_W = TOTAL // NW
CHUNK = 3328
NCH = PER_W // CHUNK
NV = CHUNK // 16
NG = NW * NCH

VR = V // 4
NT_FULL = V // 128
T_LAST = NT_FULL - 1
NI = 636
TAILR = V - NT_FULL * 128


def _relayout_tile(tsrc, tstage, iota16):
    @plsc.parallel_loop(0, 32, unroll=8)
    def _row(R):
        for L0 in range(0, 128, 16):
            d_vec = iota16 + (L0 % 32)
            rl_vec = jnp.full((16,), L0 // 32, jnp.int32) + 4 * R
            tstage[R, pl.ds(L0, 16)] = plsc.load_gather(tsrc, [d_vec, rl_vec])


def _k1_body(tT_hbm, tail_hbm, out_hbm, tsrc0, tsrc1, tst0, tst1, si0, si1, so0, so1):
    wid = lax.axis_index("s") * NC + lax.axis_index("c")
    iota16 = lax.iota(jnp.int32, 16)
    tsrc = (tsrc0, tsrc1)
    tst = (tst0, tst1)
    sin = (si0, si1)
    sout = (so0, so1)

    def tile_of(i):
        return jnp.minimum(wid + 32 * i, T_LAST)

    def issue_in(i, b):
        t = tile_of(i)
        pltpu.async_copy(
            tT_hbm.at[:, pl.ds(128 * t, 128)], tsrc[b], sin[b]
        )

    def wait_in(b):
        pltpu.make_async_copy(
            tT_hbm.at[:, pl.ds(0, 128)], tsrc[b], sin[b]
        ).wait()

    def issue_out(i, b):
        t = tile_of(i)
        pltpu.async_copy(tst[b], out_hbm.at[pl.ds(32 * t, 32)], sout[b])

    def wait_out(b):
        pltpu.make_async_copy(
            tst[b], out_hbm.at[pl.ds(0, 32)], sout[b]
        ).wait()

    issue_in(0, 0)
    issue_in(1, 1)
    for b in (0, 1):
        wait_in(b)
        _relayout_tile(tsrc[b], tst[b], iota16)
        issue_out(b, b)
        issue_in(b + 2, b)

    def pair_body(j):
        for b in (0, 1):
            i = 2 * j + b
            wait_in(b)
            wait_out(b)
            _relayout_tile(tsrc[b], tst[b], iota16)
            issue_out(i, b)
            issue_in(i + 2, b)

    pl.loop(1, NI // 2)(pair_body)

    for b in (0, 1):
        wait_in(b)
        wait_out(b)

    @pl.when(wid == 0)
    def _tail():
        pltpu.sync_copy(tail_hbm, out_hbm.at[pl.ds(32 * NT_FULL, TAILR // 4)])


@jax.jit
def _relayout_call(tableT, tail16):
    mesh = plsc.VectorSubcoreMesh(
        core_axis_name="c", subcore_axis_name="s", num_cores=NC, num_subcores=NS
    )
    return pl.kernel(
        _k1_body,
        out_type=jax.ShapeDtypeStruct((VR, 128), jnp.float32),
        mesh=mesh,
        scratch_types=[
            pltpu.VMEM((32, 128), jnp.float32),
            pltpu.VMEM((32, 128), jnp.float32),
            pltpu.VMEM((32, 128), jnp.float32),
            pltpu.VMEM((32, 128), jnp.float32),
            pltpu.SemaphoreType.DMA,
            pltpu.SemaphoreType.DMA,
            pltpu.SemaphoreType.DMA,
            pltpu.SemaphoreType.DMA,
        ],
        compiler_params=pltpu.CompilerParams(
            use_tc_tiling_on_sc=True, needs_layout_passes=False
        ),
    )(tableT, tail16)


def _emb_body(x_hbm, off_hbm, table_hbm, out_hbm, obuf, xbuf, rows, sem):
    wid = lax.axis_index("s") * NC + lax.axis_index("c")
    pltpu.sync_copy(off_hbm, obuf)

    def chunk_body(c):
        g = wid * NCH + c
        pltpu.sync_copy(x_hbm.at[g], xbuf)
        for v in range(NV):
            sl = pl.ds(v * 16, 16)
            xbuf[sl] = xbuf[sl] + obuf[sl]
        pltpu.async_copy(table_hbm.at[xbuf], rows, sem).wait()
        pltpu.sync_copy(rows, out_hbm.at[g])

    pl.loop(0, NCH)(chunk_body)


@jax.jit
def _emb_call(x2, off1, table):
    mesh = plsc.VectorSubcoreMesh(
        core_axis_name="c", subcore_axis_name="s", num_cores=NC, num_subcores=NS
    )
    return pl.kernel(
        _emb_body,
        out_type=jax.ShapeDtypeStruct((NG, CHUNK, D), jnp.float32),
        mesh=mesh,
        scratch_types=[
            pltpu.VMEM((CHUNK,), jnp.int32),
            pltpu.VMEM((CHUNK,), jnp.int32),
            pltpu.VMEM((CHUNK, D), jnp.float32),
            pltpu.SemaphoreType.DMA,
        ],
        compiler_params=pltpu.CompilerParams(use_tc_tiling_on_sc=False),
    )(x2, off1, table)


def kernel(x, table, offsets):
    tail16 = table[V - TAILR:, :].reshape(TAILR // 4, 128)
    t128 = _relayout_call(table.T, tail16)
    tableR = t128.reshape(V, D)
    x2 = x.reshape(NG, CHUNK)
    off1 = jnp.tile(offsets, CHUNK // F)
    out = _emb_call(x2, off1, tableR)
    return out.reshape(B, F * D)
